# baseline (speedup 1.0000x reference)
import functools

import jax
import jax.numpy as jnp
from jax import lax
from jax.experimental import pallas as pl
from jax.experimental.pallas import tpu as pltpu
from jax.experimental.pallas import tpu_sc as plsc

VOCAB = 1000000
DIM = 64
PADW = 128
B_TOTAL = 4096 * 200

_NC = 2
_NS = 16
NW = _NC * _NS
BPW = B_TOTAL // NW
CHUNK = 800
NSTEPS = BPW // CHUNK
NBUF = 2
NGROUPS = NSTEPS // NBUF

_mesh = plsc.VectorSubcoreMesh(core_axis_name="c", subcore_axis_name="s")

GB = 32768
SB = GB // 2
NBLK = (VOCAB + GB - 1) // GB
SROWS = NBLK * SB
TROWS = 2 * SROWS


@functools.partial(
    pl.pallas_call,
    grid=(NBLK,),
    in_specs=[pl.BlockSpec((DIM, GB), lambda j: (0, j))],
    out_specs=pl.BlockSpec((SB, 2 * DIM), lambda j: (j, 0)),
    out_shape=jax.ShapeDtypeStruct((SROWS, 2 * DIM), jnp.float32),
)
def _detile_table(t_ref, o_ref):
    t = t_ref[...]
    o_ref[:, 0:DIM] = t[:, 0:SB].T
    o_ref[:, DIM:2 * DIM] = t[:, SB:GB].T


@functools.partial(
    pl.kernel,
    mesh=_mesh,
    compiler_params=pltpu.CompilerParams(use_tc_tiling_on_sc=False),
    out_type=jax.ShapeDtypeStruct((B_TOTAL, PADW), jnp.float32),
    scratch_types=[
        pltpu.VMEM((NBUF, CHUNK), jnp.int32),
        pltpu.VMEM((NBUF, CHUNK, DIM), jnp.float32),
        pltpu.SemaphoreType.DMA,
        pltpu.SemaphoreType.DMA,
        pltpu.SemaphoreType.DMA,
        pltpu.SemaphoreType.DMA,
        pltpu.SemaphoreType.DMA,
        pltpu.SemaphoreType.DMA,
    ],
)
def _gather(idx_hbm, table_hbm, out_hbm, idx_v, rows_v,
            is0, is1, gs0, gs1, os0, os1):
    isem = (is0, is1)
    gsem = (gs0, gs1)
    osem = (os0, os1)
    wid = lax.axis_index("s") * _NC + lax.axis_index("c")
    base = wid * BPW

    def idx_copy(b, s):
        return pltpu.make_async_copy(
            idx_hbm.at[pl.ds(base + s * CHUNK, CHUNK)], idx_v.at[b], isem[b])

    def gather_copy(b):
        return pltpu.make_async_copy(
            table_hbm.at[idx_v.at[b]], rows_v.at[b], gsem[b])

    def remap(b):
        for k in range(CHUNK // 16):
            sl = pl.ds(16 * k, 16)
            i = idx_v.at[b][sl]
            l = lax.bitwise_and(i, GB - 1)
            d2 = l + l
            idx_v.at[b][sl] = (i - l) + jnp.where(l < SB, d2, d2 - (GB - 1))

    def out_copy(b, s):
        return pltpu.make_async_copy(
            rows_v.at[b],
            out_hbm.at[pl.ds(base + s * CHUNK, CHUNK), pl.ds(0, DIM)],
            osem[b])

    for b in range(NBUF):
        idx_copy(b, b).start()
    for b in range(NBUF):
        idx_copy(b, b).wait()
        remap(b)
        gather_copy(b).start()

    def group(g, carry):
        for b in range(NBUF):
            s = g * NBUF + b
            gather_copy(b).wait()
            out_copy(b, s).start()
            idx_copy(b, s + NBUF).start()
        for b in range(NBUF):
            s = g * NBUF + b
            idx_copy(b, s + NBUF).wait()
            remap(b)
            out_copy(b, s).wait()
            gather_copy(b).start()
        return carry

    lax.fori_loop(0, NGROUPS - 1, group, 0)

    for b in range(NBUF):
        s = (NGROUPS - 1) * NBUF + b
        gather_copy(b).wait()
        out_copy(b, s).start()
    for b in range(NBUF):
        s = (NGROUPS - 1) * NBUF + b
        out_copy(b, s).wait()


def kernel(x, input_embedding):
    idx = x.reshape(-1).astype(jnp.int32)
    tbl = _detile_table(input_embedding.T).reshape(TROWS, DIM)
    out = _gather(idx, tbl)
    return out[:, :DIM].reshape(x.shape[0], x.shape[1], DIM)


# speedup vs baseline: 1.0050x; 1.0050x over previous
import functools

import jax
import jax.numpy as jnp
from jax import lax
from jax.experimental import pallas as pl
from jax.experimental.pallas import tpu as pltpu
from jax.experimental.pallas import tpu_sc as plsc

VOCAB = 1000000
DIM = 64
PADW = 128
B_TOTAL = 4096 * 200

_NC = 2
_NS = 16
NW = _NC * _NS
BPW = B_TOTAL // NW
CHUNK = 400
NSTEPS = BPW // CHUNK
NBUF = 4
NGROUPS = NSTEPS // NBUF

_mesh = plsc.VectorSubcoreMesh(core_axis_name="c", subcore_axis_name="s")

GB = 32768
SB = GB // 2
NBLK = (VOCAB + GB - 1) // GB
SROWS = NBLK * SB
TROWS = 2 * SROWS


@functools.partial(
    pl.pallas_call,
    grid=(NBLK,),
    in_specs=[pl.BlockSpec((DIM, GB), lambda j: (0, j))],
    out_specs=pl.BlockSpec((SB, 2 * DIM), lambda j: (j, 0)),
    out_shape=jax.ShapeDtypeStruct((SROWS, 2 * DIM), jnp.float32),
)
def _detile_table(t_ref, o_ref):
    t = t_ref[...]
    o_ref[:, 0:DIM] = t[:, 0:SB].T
    o_ref[:, DIM:2 * DIM] = t[:, SB:GB].T


@functools.partial(
    pl.kernel,
    mesh=_mesh,
    compiler_params=pltpu.CompilerParams(use_tc_tiling_on_sc=False),
    out_type=jax.ShapeDtypeStruct((B_TOTAL, PADW), jnp.float32),
    scratch_types=[
        pltpu.VMEM((NBUF, CHUNK), jnp.int32),
        pltpu.VMEM((NBUF, CHUNK, DIM), jnp.float32),
    ] + [pltpu.SemaphoreType.DMA] * (3 * NBUF),
)
def _gather(idx_hbm, table_hbm, out_hbm, idx_v, rows_v, *sems):
    isem = sems[0:NBUF]
    gsem = sems[NBUF:2 * NBUF]
    osem = sems[2 * NBUF:3 * NBUF]
    wid = lax.axis_index("s") * _NC + lax.axis_index("c")
    base = wid * BPW

    def idx_copy(b, s):
        return pltpu.make_async_copy(
            idx_hbm.at[pl.ds(base + s * CHUNK, CHUNK)], idx_v.at[b], isem[b])

    def gather_copy(b):
        return pltpu.make_async_copy(
            table_hbm.at[idx_v.at[b]], rows_v.at[b], gsem[b])

    def remap(b):
        for k in range(CHUNK // 16):
            sl = pl.ds(16 * k, 16)
            i = idx_v.at[b][sl]
            l = lax.bitwise_and(i, GB - 1)
            d2 = l + l
            idx_v.at[b][sl] = (i - l) + jnp.where(l < SB, d2, d2 - (GB - 1))

    def out_copy(b, s):
        return pltpu.make_async_copy(
            rows_v.at[b],
            out_hbm.at[pl.ds(base + s * CHUNK, CHUNK), pl.ds(0, DIM)],
            osem[b])

    for b in range(NBUF):
        idx_copy(b, b).start()
    for b in range(NBUF):
        idx_copy(b, b).wait()
        remap(b)
        gather_copy(b).start()

    def group(g, carry):
        for b in range(NBUF):
            s = g * NBUF + b
            gather_copy(b).wait()
            out_copy(b, s).start()
            idx_copy(b, s + NBUF).start()
        for b in range(NBUF):
            s = g * NBUF + b
            idx_copy(b, s + NBUF).wait()
            remap(b)
            out_copy(b, s).wait()
            gather_copy(b).start()
        return carry

    lax.fori_loop(0, NGROUPS - 1, group, 0)

    for b in range(NBUF):
        s = (NGROUPS - 1) * NBUF + b
        gather_copy(b).wait()
        out_copy(b, s).start()
    for b in range(NBUF):
        s = (NGROUPS - 1) * NBUF + b
        out_copy(b, s).wait()


def kernel(x, input_embedding):
    idx = x.reshape(-1).astype(jnp.int32)
    tbl = _detile_table(input_embedding.T).reshape(TROWS, DIM)
    out = _gather(idx, tbl)
    return out[:, :DIM].reshape(x.shape[0], x.shape[1], DIM)
